# Initial kernel scaffold; baseline (speedup 1.0000x reference)
import functools

import jax
import jax.numpy as jnp
from jax import lax
from jax.experimental import pallas as pl
from jax.experimental.pallas import tpu as pltpu
from jax.experimental.pallas import tpu_sc as plsc

_NUM_CORES = 2
_NUM_SUBCORES = 16
_NUM_WORKERS = _NUM_CORES * _NUM_SUBCORES

_IDX_W = 128
_FIRE = 8
_CHUNK = _FIRE * _IDX_W


def _matmul_body(t_ref, w_ref, b_ref, o_ref):
    o_ref[...] = (
        lax.dot_general(
            t_ref[...], w_ref[...], (((1,), (1,)), ((), ())),
            preferred_element_type=jnp.float32,
        )
        + b_ref[...]
    )


def _transform_table(table, w, bias, blk):
    v, d = table.shape
    return pl.pallas_call(
        _matmul_body,
        grid=(v // blk,),
        in_specs=[
            pl.BlockSpec((blk, d), lambda i: (i, 0)),
            pl.BlockSpec((d, d), lambda i: (0, 0)),
            pl.BlockSpec((1, d), lambda i: (0, 0)),
        ],
        out_specs=pl.BlockSpec((blk, d), lambda i: (i, 0)),
        out_shape=jax.ShapeDtypeStruct((v, d), jnp.float32),
    )(table, w, bias.reshape(1, d))


def _make_gather(n_rows, d):
    assert n_rows % (_NUM_WORKERS * _CHUNK) == 0
    per_w = n_rows // _NUM_WORKERS
    n_steps = per_w // _CHUNK
    mesh = plsc.VectorSubcoreMesh(
        core_axis_name="c", subcore_axis_name="s",
        num_cores=_NUM_CORES, num_subcores=_NUM_SUBCORES,
    )

    @functools.partial(
        pl.kernel,
        mesh=mesh,
        out_type=jax.ShapeDtypeStruct((n_rows, d), jnp.float32),
        scratch_types=[
            pltpu.VMEM((_FIRE, _IDX_W), jnp.int32),
            pltpu.VMEM((_CHUNK, d), jnp.float32),
            pltpu.SemaphoreType.DMA,
        ],
    )
    def gather_kernel(ttab_hbm, idx_hbm, out_hbm, idx_v, rows_v, sem):
        wid = lax.axis_index("s") * _NUM_CORES + lax.axis_index("c")
        base = wid * per_w

        def step(i, carry):
            off = base + i * _CHUNK
            pltpu.sync_copy(
                idx_hbm.at[pl.ds(off // _IDX_W, _FIRE)], idx_v
            )
            copies = []
            for j in range(_FIRE):
                copies.append(
                    pltpu.async_copy(
                        ttab_hbm.at[idx_v.at[j]],
                        rows_v.at[pl.ds(j * _IDX_W, _IDX_W)],
                        sem,
                    )
                )
            for c in copies:
                c.wait()
            pltpu.sync_copy(rows_v, out_hbm.at[pl.ds(off, _CHUNK)])
            return carry

        lax.fori_loop(0, n_steps, step, 0, unroll=False)

    return gather_kernel


def kernel(x, table, W, b):
    v, d = table.shape
    bsz, seq = x.shape
    n_rows = bsz * seq

    ttable = _transform_table(table, W, b, blk=8000)
    idx2d = x.reshape(n_rows // _IDX_W, _IDX_W)
    out = _make_gather(n_rows, d)(ttable, idx2d)
    return out.reshape(bsz, seq, d)


# speedup vs baseline: 1.3288x; 1.3288x over previous
import functools

import jax
import jax.numpy as jnp
from jax import lax
from jax.experimental import pallas as pl
from jax.experimental.pallas import tpu as pltpu
from jax.experimental.pallas import tpu_sc as plsc

_NUM_CORES = 2
_NUM_SUBCORES = 16
_NUM_WORKERS = _NUM_CORES * _NUM_SUBCORES

_IDX_W = 128
_FIRE = 8
_CHUNK = _FIRE * _IDX_W


def _matmul_body(t_ref, w_ref, b_ref, o_ref):
    o_ref[...] = (
        lax.dot_general(
            t_ref[...], w_ref[...], (((1,), (1,)), ((), ())),
            preferred_element_type=jnp.float32,
        )
        + b_ref[...]
    )


def _transform_table(table, w, bias, blk):
    v, d = table.shape
    return pl.pallas_call(
        _matmul_body,
        grid=(v // blk,),
        in_specs=[
            pl.BlockSpec((blk, d), lambda i: (i, 0)),
            pl.BlockSpec((d, d), lambda i: (0, 0)),
            pl.BlockSpec((1, d), lambda i: (0, 0)),
        ],
        out_specs=pl.BlockSpec((blk, d), lambda i: (i, 0)),
        out_shape=jax.ShapeDtypeStruct((v, d), jnp.float32),
    )(table, w, bias.reshape(1, d))


def _make_gather(n_rows, d):
    assert n_rows % (_NUM_WORKERS * _CHUNK) == 0
    per_w = n_rows // _NUM_WORKERS
    n_steps = per_w // _CHUNK
    mesh = plsc.VectorSubcoreMesh(
        core_axis_name="c", subcore_axis_name="s",
        num_cores=_NUM_CORES, num_subcores=_NUM_SUBCORES,
    )

    @functools.partial(
        pl.kernel,
        mesh=mesh,
        out_type=jax.ShapeDtypeStruct((n_rows, d), jnp.float32),
        scratch_types=[
            pltpu.VMEM((_FIRE, _IDX_W), jnp.int32),
            pltpu.VMEM((_CHUNK, d), jnp.float32),
            pltpu.SemaphoreType.DMA,
        ],
        compiler_params=pltpu.CompilerParams(use_tc_tiling_on_sc=False),
    )
    def gather_kernel(ttab_hbm, idx_hbm, out_hbm, idx_v, rows_v, sem):
        wid = lax.axis_index("s") * _NUM_CORES + lax.axis_index("c")
        base = wid * per_w

        def step(i, carry):
            off = pl.multiple_of(base + i * _CHUNK, _CHUNK)
            pltpu.sync_copy(
                idx_hbm.at[pl.ds(pl.multiple_of(off // _IDX_W, _FIRE), _FIRE)],
                idx_v,
            )
            copies = []
            for j in range(_FIRE):
                copies.append(
                    pltpu.async_copy(
                        ttab_hbm.at[idx_v.at[j]],
                        rows_v.at[pl.ds(j * _IDX_W, _IDX_W)],
                        sem,
                    )
                )
            for c in copies:
                c.wait()
            pltpu.sync_copy(rows_v, out_hbm.at[pl.ds(off, _CHUNK)])
            return carry

        lax.fori_loop(0, n_steps, step, 0, unroll=False)

    return gather_kernel


def kernel(x, table, W, b):
    v, d = table.shape
    bsz, seq = x.shape
    n_rows = bsz * seq

    ttable = _transform_table(table, W, b, blk=8000)
    idx2d = x.reshape(n_rows // _IDX_W, _IDX_W)
    out = _make_gather(n_rows, d)(ttable, idx2d)
    return out.reshape(bsz, seq, d)


# speedup vs baseline: 1.6879x; 1.2703x over previous
import functools

import jax
import jax.numpy as jnp
from jax import lax
from jax.experimental import pallas as pl
from jax.experimental.pallas import tpu as pltpu
from jax.experimental.pallas import tpu_sc as plsc

_NUM_CORES = 2
_NUM_SUBCORES = 16
_NUM_WORKERS = _NUM_CORES * _NUM_SUBCORES

_LANES = 16
_BB = 8
_LPAD = 128


def _matmul_body(t_ref, w_ref, b_ref, o_ref):
    o_ref[...] = (
        lax.dot_general(
            t_ref[...], w_ref[...], (((1,), (1,)), ((), ())),
            preferred_element_type=jnp.float32,
        )
        + b_ref[...]
    )


def _transform_table(table, w2, b2, blk):
    v, d = table.shape
    dd = w2.shape[0]
    return pl.pallas_call(
        _matmul_body,
        grid=(v // blk,),
        in_specs=[
            pl.BlockSpec((blk, d), lambda i: (i, 0)),
            pl.BlockSpec((dd, d), lambda i: (0, 0)),
            pl.BlockSpec((1, dd), lambda i: (0, 0)),
        ],
        out_specs=pl.BlockSpec((blk, dd), lambda i: (i, 0)),
        out_shape=jax.ShapeDtypeStruct((v, dd), jnp.float32),
    )(table, w2, b2.reshape(1, dd))


def _make_gather(bsz, seq, d):
    assert bsz % (_NUM_WORKERS * _BB) == 0
    per_w = bsz // _NUM_WORKERS
    n_slabs = per_w // _BB
    mesh = plsc.VectorSubcoreMesh(
        core_axis_name="c", subcore_axis_name="s",
        num_cores=_NUM_CORES, num_subcores=_NUM_SUBCORES,
    )

    @functools.partial(
        pl.kernel,
        mesh=mesh,
        out_type=jax.ShapeDtypeStruct((bsz, seq, d), jnp.float32),
        scratch_types=[
            pltpu.VMEM((_BB, _LPAD), jnp.int32),
            [pltpu.VMEM((seq, 2 * d), jnp.float32) for _ in range(_BB)],
            pltpu.VMEM((_BB, seq, d), jnp.float32),
            pltpu.SemaphoreType.DMA,
        ],
        compiler_params=pltpu.CompilerParams(use_tc_tiling_on_sc=True),
    )
    def gather_kernel(ttab_hbm, xpad_hbm, out_hbm, idx_v, rows, slab_v, sem):
        wid = lax.axis_index("s") * _NUM_CORES + lax.axis_index("c")
        e0 = wid * per_w

        def slab(s, carry):
            b0 = pl.multiple_of(e0 + s * _BB, _BB)
            pltpu.sync_copy(xpad_hbm.at[pl.ds(b0, _BB)], idx_v)
            copies = []
            for j in range(_BB):
                copies.append(
                    pltpu.async_copy(
                        ttab_hbm.at[idx_v.at[j, pl.ds(0, seq)]],
                        rows[j],
                        sem,
                    )
                )
            for j in range(_BB):
                copies[j].wait()

                def compact(t, c, j=j):
                    for k in range(d // _LANES):
                        slab_v[j, t, pl.ds(k * _LANES, _LANES)] = (
                            rows[j][t, pl.ds(k * _LANES, _LANES)]
                        )
                    return c

                lax.fori_loop(0, seq, compact, 0, unroll=False)
            pltpu.sync_copy(slab_v, out_hbm.at[pl.ds(b0, _BB)])
            return carry

        lax.fori_loop(0, n_slabs, slab, 0, unroll=False)

    return gather_kernel


def kernel(x, table, W, b):
    v, d = table.shape
    bsz, seq = x.shape

    w2 = jnp.concatenate([W, W], axis=0)
    b2 = jnp.concatenate([b, b], axis=0)
    ttable = _transform_table(table, w2, b2, blk=8000)

    xpad = jnp.pad(x, ((0, 0), (0, _LPAD - seq)))
    return _make_gather(bsz, seq, d)(ttable, xpad)
